# baseline (speedup 1.0000x reference)
import functools

import jax
import jax.numpy as jnp
from jax import lax
from jax.experimental import pallas as pl
from jax.experimental.pallas import tpu as pltpu
from jax.experimental.pallas import tpu_sc as plsc

N = 10000
E = 320000
G = 16
ALPHA = 10.0

NC = 2
NS = 16
NW = NC * NS
EW = E // NW
C = 80
NCHUNK = EW // C
RPS = 624
TAIL = N - NS * RPS
ZR = 16

_LP = (3.50755205e-06, 0.999792436, -0.496977911, 0.314590535,
       -0.188782674, 0.0817268084, -0.0172080611)


def _act(f, s):
    tf = jnp.exp(-jnp.abs(f))
    num = jnp.where(f >= 0.0, jnp.float32(1.0), tf)
    sig = num / (1.0 + tf)
    t = jnp.exp(-jnp.abs(s))
    p = jnp.float32(_LP[6])
    for co in _LP[5::-1]:
        p = p * t + jnp.float32(co)
    return sig * (jnp.maximum(s, 0.0) + p)


def _edge_body(td_hbm, ts_hbm, q_hbm, dst_hbm, src_hbm, out_hbm,
               idx_d, idx_s, rows_d, rows_s, qbuf, msg, zbuf, aggr_sh,
               sem_d, sem_s, sem_q):
    c = lax.axis_index("c")
    s = lax.axis_index("s")
    wid = s * NC + c
    zero16 = jnp.zeros((16,), jnp.float32)
    lane0 = jnp.where(lax.iota(jnp.int32, 16) == 0,
                      jnp.float32(1.0), jnp.float32(0.0))

    def zrow(r, _):
        for j in range(8):
            zbuf[r, pl.ds(j * 16, 16)] = zero16
        return 0
    lax.fori_loop(0, ZR, zrow, 0)

    def mrow(r, _):
        for j in range(4, 8):
            msg[r, pl.ds(j * 16, 16)] = lane0 if j == 4 else zero16
        return 0
    lax.fori_loop(0, C, mrow, 0)

    def zcp(k, _):
        pltpu.sync_copy(zbuf, aggr_sh.at[pl.ds(s * RPS + k * ZR, ZR)])
        return 0
    lax.fori_loop(0, RPS // ZR, zcp, 0)

    @pl.when(s == NS - 1)
    def _():
        pltpu.sync_copy(zbuf, aggr_sh.at[pl.ds(NS * RPS, TAIL)])
    plsc.subcore_barrier()

    base0 = wid * EW

    def chunk(tt, _):
        base = base0 + tt * C
        pltpu.sync_copy(dst_hbm.at[pl.ds(base, C)], idx_d)
        pltpu.sync_copy(src_hbm.at[pl.ds(base, C)], idx_s)
        cp_d = pltpu.async_copy(td_hbm.at[idx_d], rows_d, sem_d)
        cp_s = pltpu.async_copy(ts_hbm.at[idx_s], rows_s, sem_s)
        cp_q = pltpu.async_copy(q_hbm.at[pl.ds(base, C)], qbuf, sem_q)
        cp_d.wait()
        cp_s.wait()
        cp_q.wait()

        @plsc.parallel_loop(0, C, unroll=2)
        def edge(e):
            for j in range(4):
                slf = pl.ds(j * 16, 16)
                sls = pl.ds(64 + j * 16, 16)
                f = rows_d[e, slf] + rows_s[e, slf] + qbuf[e, slf]
                sv = rows_d[e, sls] + rows_s[e, sls] + qbuf[e, sls]
                msg[e, slf] = _act(f, sv)
        pltpu.sync_copy(msg, aggr_sh.at[idx_d], add=True)
        return 0
    lax.fori_loop(0, NCHUNK, chunk, 0)

    plsc.subcore_barrier()
    sl = pl.ds(s * RPS, RPS)
    pltpu.sync_copy(aggr_sh.at[sl], out_hbm.at[c, sl])

    @pl.when(s == NS - 1)
    def _():
        tl = pl.ds(NS * RPS, TAIL)
        pltpu.sync_copy(aggr_sh.at[tl], out_hbm.at[c, tl])


_sc_mesh = plsc.VectorSubcoreMesh(core_axis_name="c", subcore_axis_name="s")

_edge_call = pl.kernel(
    _edge_body,
    out_type=jax.ShapeDtypeStruct((NC, N, 128), jnp.float32),
    mesh=_sc_mesh,
    scratch_types=[
        pltpu.VMEM((C,), jnp.int32),
        pltpu.VMEM((C,), jnp.int32),
        pltpu.VMEM((C, 128), jnp.float32),
        pltpu.VMEM((C, 128), jnp.float32),
        pltpu.VMEM((C, 128), jnp.float32),
        pltpu.VMEM((C, 128), jnp.float32),
        pltpu.VMEM((ZR, 128), jnp.float32),
        pltpu.VMEM_SHARED((N, 128), jnp.float32),
        pltpu.SemaphoreType.DMA,
        pltpu.SemaphoreType.DMA,
        pltpu.SemaphoreType.DMA,
    ],
    name="cgcnn_edge_stage",
)


def kernel(x, edge_index, edge_attr, edge_dist, batch, r_min_raw, r_delta_raw, W_pre, b_pre, Wf0, bf0, Ws0, bs0, gam0, bet0, Wf1, bf1, Ws1, bs1, gam1, bet1, Wf2, bf2, Ws2, bs2, gam2, bet2, W_post, b_post, W_out, b_out):
    sp = lambda v: jnp.logaddexp(v, 0.0)
    r_min = sp(r_min_raw)
    r_max = r_min + sp(r_delta_raw)
    dist = edge_dist.reshape(-1, 1)
    gate = jax.nn.sigmoid(ALPHA * (dist - r_min)) * jax.nn.sigmoid(ALPHA * (r_max - dist))
    ea = edge_attr * gate
    src = edge_index[0]
    dst = edge_index[1]

    out = jax.nn.relu(x @ W_pre + b_pre)
    deg = None
    layers = ((Wf0, bf0, Ws0, bs0, gam0, bet0),
              (Wf1, bf1, Ws1, bs1, gam1, bet1),
              (Wf2, bf2, Ws2, bs2, gam2, bet2))
    for (Wf, bf, Ws, bs, gam, bet) in layers:
        Td = out @ jnp.concatenate([Wf[:64], Ws[:64]], axis=1)
        Ts = out @ jnp.concatenate([Wf[64:128], Ws[64:128]], axis=1)
        Q = ea @ jnp.concatenate([Wf[128:], Ws[128:]], axis=1) + jnp.concatenate([bf, bs])
        partials = _edge_call(Td, Ts, Q, dst, src)
        acc = partials[0] + partials[1]
        if deg is None:
            deg = jnp.maximum(acc[:, 64], 1.0)
        aggr = acc[:, :64] / deg[:, None]
        h = out + aggr
        mu = h.mean(axis=0)
        var = h.var(axis=0)
        out = (h - mu) / jnp.sqrt(var + 1e-5) * gam + bet

    cnt = jnp.maximum(jax.ops.segment_sum(jnp.ones((N,), jnp.float32), batch, num_segments=G), 1.0)
    pooled = jax.ops.segment_sum(out, batch, num_segments=G) / cnt[:, None]
    emb = jax.nn.relu(pooled @ W_post + b_post)
    return emb @ W_out + b_out


# speedup vs baseline: 1.3058x; 1.3058x over previous
import functools

import jax
import jax.numpy as jnp
from jax import lax
from jax.experimental import pallas as pl
from jax.experimental.pallas import tpu as pltpu
from jax.experimental.pallas import tpu_sc as plsc

N = 10000
E = 320000
G = 16
ALPHA = 10.0

NC = 2
NS = 16
NW = NC * NS
EW = E // NW
C = 40
NCHUNK = EW // C
RPS = 624
TAIL = N - NS * RPS
ZR = 16

_LP = (3.50755205e-06, 0.999792436, -0.496977911, 0.314590535,
       -0.188782674, 0.0817268084, -0.0172080611)


def _act(f, s):
    tf = jnp.exp(-jnp.abs(f))
    num = jnp.where(f >= 0.0, jnp.float32(1.0), tf)
    sig = num / (1.0 + tf)
    t = jnp.exp(-jnp.abs(s))
    p = jnp.float32(_LP[6])
    for co in _LP[5::-1]:
        p = p * t + jnp.float32(co)
    return sig * (jnp.maximum(s, 0.0) + p)


def _edge_body(td_hbm, ts_hbm, q_hbm, dst_hbm, src_hbm, out_hbm,
               idx_d0, idx_d1, idx_s0, idx_s1, idx_c0, idx_c1,
               rd0, rd1, rs0, rs1, qb0, qb1, mb0, mb1, zbuf, aggr_sh,
               sd0, sd1, ss0, ss1, sq0, sq1, sc0, sc1):
    c = lax.axis_index("c")
    s = lax.axis_index("s")
    wid = s * NC + c
    zero16 = jnp.zeros((16,), jnp.float32)
    lane0 = jnp.where(lax.iota(jnp.int32, 16) == 0,
                      jnp.float32(1.0), jnp.float32(0.0))

    idx_d = (idx_d0, idx_d1)
    idx_s = (idx_s0, idx_s1)
    idx_c = (idx_c0, idx_c1)
    rows_d = (rd0, rd1)
    rows_s = (rs0, rs1)
    qbuf = (qb0, qb1)
    msg = (mb0, mb1)
    sem_d = (sd0, sd1)
    sem_s = (ss0, ss1)
    sem_q = (sq0, sq1)
    sem_c = (sc0, sc1)

    def zrow(r, _):
        for j in range(8):
            zbuf[r, pl.ds(j * 16, 16)] = zero16
        return 0
    lax.fori_loop(0, ZR, zrow, 0)

    def mrow(r, _):
        for b in range(2):
            for j in range(4, 8):
                msg[b][r, pl.ds(j * 16, 16)] = lane0 if j == 4 else zero16
        return 0
    lax.fori_loop(0, C, mrow, 0)

    def zcp(k, _):
        pltpu.sync_copy(zbuf, aggr_sh.at[pl.ds(s * RPS + k * ZR, ZR)])
        return 0
    lax.fori_loop(0, RPS // ZR, zcp, 0)

    @pl.when(s == NS - 1)
    def _():
        pltpu.sync_copy(zbuf, aggr_sh.at[pl.ds(NS * RPS, TAIL)])
    plsc.subcore_barrier()

    base0 = wid * EW

    def fire(tt, b):
        base = base0 + tt * C
        pltpu.sync_copy(dst_hbm.at[pl.ds(base, C)], idx_d[b])
        pltpu.sync_copy(src_hbm.at[pl.ds(base, C)], idx_s[b])
        pltpu.async_copy(td_hbm.at[idx_d[b]], rows_d[b], sem_d[b])
        pltpu.async_copy(ts_hbm.at[idx_s[b]], rows_s[b], sem_s[b])
        pltpu.async_copy(q_hbm.at[pl.ds(base, C)], qbuf[b], sem_q[b])

    def wait_gathers(b):
        pltpu.make_async_copy(td_hbm.at[idx_d[b]], rows_d[b], sem_d[b]).wait()
        pltpu.make_async_copy(ts_hbm.at[idx_s[b]], rows_s[b], sem_s[b]).wait()
        pltpu.make_async_copy(q_hbm.at[pl.ds(base0, C)], qbuf[b], sem_q[b]).wait()

    def wait_scatter(b):
        pltpu.make_async_copy(msg[b], aggr_sh.at[idx_c[b]], sem_c[b]).wait()

    fire(0, 0)

    def pair(k, _):
        for b in range(2):
            tt = k * 2 + b
            if b == 0:
                @pl.when(k >= 1)
                def _():
                    wait_scatter(0)
            else:
                @pl.when(k >= 1)
                def _():
                    wait_scatter(1)
            if b == 0:
                fire(tt + 1, 1)
            else:
                @pl.when(k < NCHUNK // 2 - 1)
                def _():
                    fire(tt + 2 - 1, 0)
            wait_gathers(b)

            for off in (0, 16, C - 16):
                idx_c[b][pl.ds(off, 16)] = idx_d[b][pl.ds(off, 16)]

            def edge(e, _):
                for j in range(4):
                    slf = pl.ds(j * 16, 16)
                    sls = pl.ds(64 + j * 16, 16)
                    f = rows_d[b][e, slf] + rows_s[b][e, slf] + qbuf[b][e, slf]
                    sv = rows_d[b][e, sls] + rows_s[b][e, sls] + qbuf[b][e, sls]
                    msg[b][e, slf] = _act(f, sv)
                return 0
            lax.fori_loop(0, C, edge, 0)
            pltpu.async_copy(msg[b], aggr_sh.at[idx_c[b]], sem_c[b], add=True)
        return 0
    lax.fori_loop(0, NCHUNK // 2, pair, 0)
    wait_scatter(0)
    wait_scatter(1)

    plsc.subcore_barrier()
    sl = pl.ds(s * RPS, RPS)
    pltpu.sync_copy(aggr_sh.at[sl], out_hbm.at[c, sl])

    @pl.when(s == NS - 1)
    def _():
        tl = pl.ds(NS * RPS, TAIL)
        pltpu.sync_copy(aggr_sh.at[tl], out_hbm.at[c, tl])


_sc_mesh = plsc.VectorSubcoreMesh(core_axis_name="c", subcore_axis_name="s")

_edge_call = pl.kernel(
    _edge_body,
    out_type=jax.ShapeDtypeStruct((NC, N, 128), jnp.float32),
    mesh=_sc_mesh,
    scratch_types=(
        [pltpu.VMEM((C,), jnp.int32)] * 6
        + [pltpu.VMEM((C, 128), jnp.float32)] * 8
        + [pltpu.VMEM((ZR, 128), jnp.float32),
           pltpu.VMEM_SHARED((N, 128), jnp.float32)]
        + [pltpu.SemaphoreType.DMA] * 8
    ),
    name="cgcnn_edge_stage",
)


def kernel(x, edge_index, edge_attr, edge_dist, batch, r_min_raw, r_delta_raw, W_pre, b_pre, Wf0, bf0, Ws0, bs0, gam0, bet0, Wf1, bf1, Ws1, bs1, gam1, bet1, Wf2, bf2, Ws2, bs2, gam2, bet2, W_post, b_post, W_out, b_out):
    sp = lambda v: jnp.logaddexp(v, 0.0)
    r_min = sp(r_min_raw)
    r_max = r_min + sp(r_delta_raw)
    dist = edge_dist.reshape(-1, 1)
    gate = jax.nn.sigmoid(ALPHA * (dist - r_min)) * jax.nn.sigmoid(ALPHA * (r_max - dist))
    ea = edge_attr * gate
    src = edge_index[0]
    dst = edge_index[1]

    out = jax.nn.relu(x @ W_pre + b_pre)
    deg = None
    layers = ((Wf0, bf0, Ws0, bs0, gam0, bet0),
              (Wf1, bf1, Ws1, bs1, gam1, bet1),
              (Wf2, bf2, Ws2, bs2, gam2, bet2))
    for (Wf, bf, Ws, bs, gam, bet) in layers:
        Td = out @ jnp.concatenate([Wf[:64], Ws[:64]], axis=1)
        Ts = out @ jnp.concatenate([Wf[64:128], Ws[64:128]], axis=1)
        Q = ea @ jnp.concatenate([Wf[128:], Ws[128:]], axis=1) + jnp.concatenate([bf, bs])
        partials = _edge_call(Td, Ts, Q, dst, src)
        acc = partials[0] + partials[1]
        if deg is None:
            deg = jnp.maximum(acc[:, 64], 1.0)
        aggr = acc[:, :64] / deg[:, None]
        h = out + aggr
        mu = h.mean(axis=0)
        var = h.var(axis=0)
        out = (h - mu) / jnp.sqrt(var + 1e-5) * gam + bet

    cnt = jnp.maximum(jax.ops.segment_sum(jnp.ones((N,), jnp.float32), batch, num_segments=G), 1.0)
    pooled = jax.ops.segment_sum(out, batch, num_segments=G) / cnt[:, None]
    emb = jax.nn.relu(pooled @ W_post + b_post)
    return emb @ W_out + b_out


# speedup vs baseline: 1.5855x; 1.2142x over previous
import functools

import jax
import jax.numpy as jnp
from jax import lax
from jax.experimental import pallas as pl
from jax.experimental.pallas import tpu as pltpu
from jax.experimental.pallas import tpu_sc as plsc

N = 10000
E = 320000
G = 16
ALPHA = 10.0

NC = 2
NS = 16
NW = NC * NS
EW = E // NW
C = 40
NCHUNK = EW // C
BLK = 10
NBLK = NCHUNK // BLK
RPS = 624
TAIL = N - NS * RPS
ZR = 16

_LP = (3.50755205e-06, 0.999792436, -0.496977911, 0.314590535,
       -0.188782674, 0.0817268084, -0.0172080611)


def _act(f, s):
    tf = jnp.exp(-jnp.abs(f))
    num = jnp.where(f >= 0.0, jnp.float32(1.0), tf)
    sig = num / (1.0 + tf)
    t = jnp.exp(-jnp.abs(s))
    p = jnp.float32(_LP[6])
    for co in _LP[5::-1]:
        p = p * t + jnp.float32(co)
    return sig * (jnp.maximum(s, 0.0) + p)


def _edge_body(td_hbm, ts_hbm, q_hbm, dst2_hbm, src2_hbm, out_hbm,
               iblk_d, iblk_s,
               rd0, rd1, rs0, rs1, qb0, qb1, mb0, mb1, zbuf, aggr_sh,
               sd0, sd1, ss0, ss1, sq0, sq1, sc0, sc1):
    c = lax.axis_index("c")
    s = lax.axis_index("s")
    wid = s * NC + c
    zero16 = jnp.zeros((16,), jnp.float32)
    lane0 = jnp.where(lax.iota(jnp.int32, 16) == 0,
                      jnp.float32(1.0), jnp.float32(0.0))

    rows_d = (rd0, rd1)
    rows_s = (rs0, rs1)
    qbuf = (qb0, qb1)
    msg = (mb0, mb1)
    sem_d = (sd0, sd1)
    sem_s = (ss0, ss1)
    sem_q = (sq0, sq1)
    sem_c = (sc0, sc1)

    def zrow(r, _):
        for j in range(8):
            zbuf[r, pl.ds(j * 16, 16)] = zero16
        return 0
    lax.fori_loop(0, ZR, zrow, 0)

    def mrow(r, _):
        for b in range(2):
            for j in range(4, 8):
                msg[b][r, pl.ds(j * 16, 16)] = lane0 if j == 4 else zero16
        return 0
    lax.fori_loop(0, C, mrow, 0)

    def zcp(k, _):
        pltpu.sync_copy(zbuf, aggr_sh.at[pl.ds(s * RPS + k * ZR, ZR)])
        return 0
    lax.fori_loop(0, RPS // ZR, zcp, 0)

    @pl.when(s == NS - 1)
    def _():
        pltpu.sync_copy(zbuf, aggr_sh.at[pl.ds(NS * RPS, TAIL)])
    plsc.subcore_barrier()

    chunk0 = wid * NCHUNK

    def fire(row, tt, b):
        pltpu.async_copy(td_hbm.at[iblk_d.at[row]], rows_d[b], sem_d[b])
        pltpu.async_copy(ts_hbm.at[iblk_s.at[row]], rows_s[b], sem_s[b])
        base = (chunk0 + tt) * C
        pltpu.async_copy(q_hbm.at[pl.ds(base, C)], qbuf[b], sem_q[b])

    def wait_gathers(row, b):
        pltpu.make_async_copy(td_hbm.at[iblk_d.at[row]], rows_d[b], sem_d[b]).wait()
        pltpu.make_async_copy(ts_hbm.at[iblk_s.at[row]], rows_s[b], sem_s[b]).wait()
        pltpu.make_async_copy(q_hbm.at[pl.ds(0, C)], qbuf[b], sem_q[b]).wait()

    def wait_scatter(row, b):
        pltpu.make_async_copy(msg[b], aggr_sh.at[iblk_d.at[row]], sem_c[b]).wait()

    def block(blk, _):
        @pl.when(blk >= 1)
        def _():
            wait_scatter(BLK - 2, 0)
            wait_scatter(BLK - 1, 1)
        pltpu.sync_copy(dst2_hbm.at[wid, blk], iblk_d)
        pltpu.sync_copy(src2_hbm.at[wid, blk], iblk_s)
        fire(0, blk * BLK, 0)

        def pair(k, _):
            for b in range(2):
                lt = k * 2 + b
                tt = blk * BLK + lt
                if b == 0:
                    @pl.when(k >= 1)
                    def _():
                        wait_scatter(lt - 2, 0)
                else:
                    @pl.when(k >= 1)
                    def _():
                        wait_scatter(lt - 2, 1)
                if b == 0:
                    fire(lt + 1, tt + 1, 1)
                else:
                    @pl.when(k < BLK // 2 - 1)
                    def _():
                        fire(lt + 1, tt + 1, 0)
                wait_gathers(lt, b)

                def edge(e, _):
                    for j in range(4):
                        slf = pl.ds(j * 16, 16)
                        sls = pl.ds(64 + j * 16, 16)
                        f = rows_d[b][e, slf] + rows_s[b][e, slf] + qbuf[b][e, slf]
                        sv = rows_d[b][e, sls] + rows_s[b][e, sls] + qbuf[b][e, sls]
                        msg[b][e, slf] = _act(f, sv)
                    return 0
                lax.fori_loop(0, C, edge, 0)
                pltpu.async_copy(msg[b], aggr_sh.at[iblk_d.at[lt]], sem_c[b], add=True)
            return 0
        lax.fori_loop(0, BLK // 2, pair, 0)
        return 0
    lax.fori_loop(0, NBLK, block, 0)
    wait_scatter(BLK - 2, 0)
    wait_scatter(BLK - 1, 1)

    plsc.subcore_barrier()
    sl = pl.ds(s * RPS, RPS)
    pltpu.sync_copy(aggr_sh.at[sl], out_hbm.at[c, sl])

    @pl.when(s == NS - 1)
    def _():
        tl = pl.ds(NS * RPS, TAIL)
        pltpu.sync_copy(aggr_sh.at[tl], out_hbm.at[c, tl])


_sc_mesh = plsc.VectorSubcoreMesh(core_axis_name="c", subcore_axis_name="s")

_edge_call = pl.kernel(
    _edge_body,
    out_type=jax.ShapeDtypeStruct((NC, N, 128), jnp.float32),
    mesh=_sc_mesh,
    scratch_types=(
        [pltpu.VMEM((BLK, C), jnp.int32)] * 2
        + [pltpu.VMEM((C, 128), jnp.float32)] * 8
        + [pltpu.VMEM((ZR, 128), jnp.float32),
           pltpu.VMEM_SHARED((N, 128), jnp.float32)]
        + [pltpu.SemaphoreType.DMA] * 8
    ),
    name="cgcnn_edge_stage",
)


def kernel(x, edge_index, edge_attr, edge_dist, batch, r_min_raw, r_delta_raw, W_pre, b_pre, Wf0, bf0, Ws0, bs0, gam0, bet0, Wf1, bf1, Ws1, bs1, gam1, bet1, Wf2, bf2, Ws2, bs2, gam2, bet2, W_post, b_post, W_out, b_out):
    sp = lambda v: jnp.logaddexp(v, 0.0)
    r_min = sp(r_min_raw)
    r_max = r_min + sp(r_delta_raw)
    dist = edge_dist.reshape(-1, 1)
    gate = jax.nn.sigmoid(ALPHA * (dist - r_min)) * jax.nn.sigmoid(ALPHA * (r_max - dist))
    ea = edge_attr * gate
    src = edge_index[0]
    dst = edge_index[1]

    dst2 = dst.reshape(NW, NBLK, BLK, C)
    src2 = src.reshape(NW, NBLK, BLK, C)

    out = jax.nn.relu(x @ W_pre + b_pre)
    deg = None
    layers = ((Wf0, bf0, Ws0, bs0, gam0, bet0),
              (Wf1, bf1, Ws1, bs1, gam1, bet1),
              (Wf2, bf2, Ws2, bs2, gam2, bet2))
    for (Wf, bf, Ws, bs, gam, bet) in layers:
        Td = out @ jnp.concatenate([Wf[:64], Ws[:64]], axis=1)
        Ts = out @ jnp.concatenate([Wf[64:128], Ws[64:128]], axis=1)
        Q = ea @ jnp.concatenate([Wf[128:], Ws[128:]], axis=1) + jnp.concatenate([bf, bs])
        partials = _edge_call(Td, Ts, Q, dst2, src2)
        acc = partials[0] + partials[1]
        if deg is None:
            deg = jnp.maximum(acc[:, 64], 1.0)
        aggr = acc[:, :64] / deg[:, None]
        h = out + aggr
        mu = h.mean(axis=0)
        var = h.var(axis=0)
        out = (h - mu) / jnp.sqrt(var + 1e-5) * gam + bet

    cnt = jnp.maximum(jax.ops.segment_sum(jnp.ones((N,), jnp.float32), batch, num_segments=G), 1.0)
    pooled = jax.ops.segment_sum(out, batch, num_segments=G) / cnt[:, None]
    emb = jax.nn.relu(pooled @ W_post + b_post)
    return emb @ W_out + b_out
